# baseline (speedup 1.0000x reference)
import functools

import jax
import jax.numpy as jnp
from jax import lax
from jax.experimental import pallas as pl
from jax.experimental.pallas import tpu as pltpu
from jax.experimental.pallas import tpu_sc as plsc

_VT = 2048
_NEG = -1e30


def _sc_gather(idx, table):
    info = plsc.get_sparse_core_info()
    nw = info.num_cores * info.num_subcores
    n = idx.shape[0]
    e = table.shape[1]
    bpw = n // nw
    mesh = plsc.VectorSubcoreMesh(core_axis_name="c", subcore_axis_name="s")

    @functools.partial(
        pl.kernel,
        mesh=mesh,
        out_type=jax.ShapeDtypeStruct((n, e), table.dtype),
        scratch_types=[
            pltpu.VMEM((bpw,), jnp.int32),
            pltpu.VMEM((bpw, e), table.dtype),
            pltpu.SemaphoreType.DMA,
        ],
    )
    def gk(idx_hbm, table_hbm, out_hbm, idx_v, rows_v, sem):
        wid = lax.axis_index("s") * info.num_cores + lax.axis_index("c")
        base = wid * bpw
        pltpu.sync_copy(idx_hbm.at[pl.ds(base, bpw)], idx_v)
        pltpu.async_copy(table_hbm.at[idx_v], rows_v, sem).wait()
        pltpu.sync_copy(rows_v, out_hbm.at[pl.ds(base, bpw)])

    return gk(idx, table)


def _logits_tile(emb_ref, w_ref, b_ref):
    acc = lax.dot_general(
        emb_ref[...].astype(jnp.bfloat16),
        w_ref[...].astype(jnp.bfloat16),
        (((1,), (1,)), ((), ())),
        preferred_element_type=jnp.float32,
    )
    return acc + b_ref[...]


def _lse_body(v, nt, emb_ref, w_ref, b_ref, lse_ref, m_acc, s_acc):
    i = pl.program_id(0)

    @pl.when(i == 0)
    def _():
        m_acc[...] = jnp.full_like(m_acc[...], _NEG)
        s_acc[...] = jnp.zeros_like(s_acc[...])

    logits = _logits_tile(emb_ref, w_ref, b_ref)
    col = i * _VT + lax.broadcasted_iota(jnp.int32, logits.shape, 1)
    logits = jnp.where(col < v, logits, _NEG)
    m_old = m_acc[...]
    m_new = jnp.maximum(m_old, jnp.max(logits, axis=1, keepdims=True))
    p_sum = jnp.sum(jnp.exp(logits - m_new), axis=1, keepdims=True)
    s_new = s_acc[...] * jnp.exp(m_old - m_new) + p_sum
    m_acc[...] = m_new
    s_acc[...] = s_new

    @pl.when(i == nt - 1)
    def _():
        lse_ref[...] = m_new + jnp.log(s_new)


def _out_body(emb_ref, w_ref, b_ref, lse_ref, y_ref):
    y_ref[...] = _logits_tile(emb_ref, w_ref, b_ref) - lse_ref[...]


def kernel(x, emb_table, W1, b1):
    b, c = x.shape
    v, e = emb_table.shape
    d = c * e
    n = b * c

    ep = max(e, 128)
    emb_pad = jnp.pad(emb_table, ((0, 0), (0, ep - e))) if ep != e else emb_table
    rows = _sc_gather(x.reshape(n).astype(jnp.int32), emb_pad)
    embeds = rows[:, :e].reshape(b, d)
    b2 = b1.reshape(1, v)
    nt = pl.cdiv(v, _VT)

    lse = pl.pallas_call(
        functools.partial(_lse_body, v, nt),
        grid=(nt,),
        in_specs=[
            pl.BlockSpec((b, d), lambda i: (0, 0)),
            pl.BlockSpec((_VT, d), lambda i: (i, 0)),
            pl.BlockSpec((1, _VT), lambda i: (0, i)),
        ],
        out_specs=pl.BlockSpec((b, 1), lambda i: (0, 0)),
        out_shape=jax.ShapeDtypeStruct((b, 1), jnp.float32),
        scratch_shapes=[
            pltpu.VMEM((b, 1), jnp.float32),
            pltpu.VMEM((b, 1), jnp.float32),
        ],
        compiler_params=pltpu.CompilerParams(
            dimension_semantics=("arbitrary",)),
    )(embeds, W1, b2)

    y = pl.pallas_call(
        _out_body,
        grid=(nt,),
        in_specs=[
            pl.BlockSpec((b, d), lambda i: (0, 0)),
            pl.BlockSpec((_VT, d), lambda i: (i, 0)),
            pl.BlockSpec((1, _VT), lambda i: (0, i)),
            pl.BlockSpec((b, 1), lambda i: (0, 0)),
        ],
        out_specs=pl.BlockSpec((b, _VT), lambda i: (0, i)),
        out_shape=jax.ShapeDtypeStruct((b, v), jnp.float32),
        compiler_params=pltpu.CompilerParams(
            dimension_semantics=("arbitrary",)),
    )(embeds, W1, b2, lse)
    return y


# speedup vs baseline: 1.0232x; 1.0232x over previous
import functools

import jax
import jax.numpy as jnp
from jax import lax
from jax.experimental import pallas as pl
from jax.experimental.pallas import tpu as pltpu
from jax.experimental.pallas import tpu_sc as plsc

_VT = 4096
_NEG = -1e30


def _sc_gather(idx, table):
    info = plsc.get_sparse_core_info()
    nw = info.num_cores * info.num_subcores
    n = idx.shape[0]
    e = table.shape[1]
    bpw = n // nw
    mesh = plsc.VectorSubcoreMesh(core_axis_name="c", subcore_axis_name="s")

    @functools.partial(
        pl.kernel,
        mesh=mesh,
        out_type=jax.ShapeDtypeStruct((n, e), table.dtype),
        scratch_types=[
            pltpu.VMEM((bpw,), jnp.int32),
            pltpu.VMEM((bpw, e), table.dtype),
            pltpu.SemaphoreType.DMA,
        ],
    )
    def gk(idx_hbm, table_hbm, out_hbm, idx_v, rows_v, sem):
        wid = lax.axis_index("s") * info.num_cores + lax.axis_index("c")
        base = wid * bpw
        pltpu.sync_copy(idx_hbm.at[pl.ds(base, bpw)], idx_v)
        pltpu.async_copy(table_hbm.at[idx_v], rows_v, sem).wait()
        pltpu.sync_copy(rows_v, out_hbm.at[pl.ds(base, bpw)])

    return gk(idx, table)


def _logits_tile(emb_ref, w_ref, b_ref):
    acc = lax.dot_general(
        emb_ref[...].astype(jnp.bfloat16),
        w_ref[...].astype(jnp.bfloat16),
        (((1,), (1,)), ((), ())),
        preferred_element_type=jnp.float32,
    )
    return acc + b_ref[...]


def _lse_body(v, nt, emb_ref, w_ref, b_ref, lse_ref, m_acc, s_acc):
    i = pl.program_id(0)

    @pl.when(i == 0)
    def _():
        m_acc[...] = jnp.full_like(m_acc[...], _NEG)
        s_acc[...] = jnp.zeros_like(s_acc[...])

    logits = _logits_tile(emb_ref, w_ref, b_ref)
    col = i * _VT + lax.broadcasted_iota(jnp.int32, logits.shape, 1)
    logits = jnp.where(col < v, logits, _NEG)
    m_old = m_acc[...]
    m_new = jnp.maximum(m_old, jnp.max(logits, axis=1, keepdims=True))
    p_sum = jnp.sum(jnp.exp(logits - m_new), axis=1, keepdims=True)
    s_new = s_acc[...] * jnp.exp(m_old - m_new) + p_sum
    m_acc[...] = m_new
    s_acc[...] = s_new

    @pl.when(i == nt - 1)
    def _():
        lse_ref[...] = m_new + jnp.log(s_new)


def _out_body(emb_ref, w_ref, b_ref, lse_ref, y_ref):
    y_ref[...] = _logits_tile(emb_ref, w_ref, b_ref) - lse_ref[...]


def kernel(x, emb_table, W1, b1):
    b, c = x.shape
    v, e = emb_table.shape
    d = c * e
    n = b * c

    ep = max(e, 128)
    emb_pad = jnp.pad(emb_table, ((0, 0), (0, ep - e))) if ep != e else emb_table
    rows = _sc_gather(x.reshape(n).astype(jnp.int32), emb_pad)
    embeds = rows[:, :e].reshape(b, d)
    b2 = b1.reshape(1, v)
    nt = pl.cdiv(v, _VT)

    lse = pl.pallas_call(
        functools.partial(_lse_body, v, nt),
        grid=(nt,),
        in_specs=[
            pl.BlockSpec((b, d), lambda i: (0, 0)),
            pl.BlockSpec((_VT, d), lambda i: (i, 0)),
            pl.BlockSpec((1, _VT), lambda i: (0, i)),
        ],
        out_specs=pl.BlockSpec((b, 1), lambda i: (0, 0)),
        out_shape=jax.ShapeDtypeStruct((b, 1), jnp.float32),
        scratch_shapes=[
            pltpu.VMEM((b, 1), jnp.float32),
            pltpu.VMEM((b, 1), jnp.float32),
        ],
        compiler_params=pltpu.CompilerParams(
            dimension_semantics=("arbitrary",)),
    )(embeds, W1, b2)

    y = pl.pallas_call(
        _out_body,
        grid=(nt,),
        in_specs=[
            pl.BlockSpec((b, d), lambda i: (0, 0)),
            pl.BlockSpec((_VT, d), lambda i: (i, 0)),
            pl.BlockSpec((1, _VT), lambda i: (0, i)),
            pl.BlockSpec((b, 1), lambda i: (0, 0)),
        ],
        out_specs=pl.BlockSpec((b, _VT), lambda i: (0, i)),
        out_shape=jax.ShapeDtypeStruct((b, v), jnp.float32),
        compiler_params=pltpu.CompilerParams(
            dimension_semantics=("arbitrary",)),
    )(embeds, W1, b2, lse)
    return y


# speedup vs baseline: 1.1264x; 1.1009x over previous
import functools

import jax
import jax.numpy as jnp
from jax import lax
from jax.experimental import pallas as pl
from jax.experimental.pallas import tpu as pltpu
from jax.experimental.pallas import tpu_sc as plsc

_VT = 4096
_NBUF = 2
_K = 8
_NEG = -1e30


def _sc_gather(idx, table):
    info = plsc.get_sparse_core_info()
    nw = info.num_cores * info.num_subcores
    n = idx.shape[0]
    e = table.shape[1]
    bpw = n // nw
    mesh = plsc.VectorSubcoreMesh(core_axis_name="c", subcore_axis_name="s")

    @functools.partial(
        pl.kernel,
        mesh=mesh,
        out_type=jax.ShapeDtypeStruct((n, e), table.dtype),
        scratch_types=[
            pltpu.VMEM((bpw,), jnp.int32),
            pltpu.VMEM((bpw, e), table.dtype),
            pltpu.SemaphoreType.DMA,
        ],
    )
    def gk(idx_hbm, table_hbm, out_hbm, idx_v, rows_v, sem):
        wid = lax.axis_index("s") * info.num_cores + lax.axis_index("c")
        base = wid * bpw
        pltpu.sync_copy(idx_hbm.at[pl.ds(base, bpw)], idx_v)
        pltpu.async_copy(table_hbm.at[idx_v], rows_v, sem).wait()
        pltpu.sync_copy(rows_v, out_hbm.at[pl.ds(base, bpw)])

    return gk(idx, table)


def _logits_tile(emb_ref, w_ref, b_ref):
    acc = lax.dot_general(
        emb_ref[...].astype(jnp.bfloat16),
        w_ref[...].astype(jnp.bfloat16),
        (((1,), (1,)), ((), ())),
        preferred_element_type=jnp.float32,
    )
    return acc + b_ref[...]


def _lse_body(v, nt, emb_ref, w_ref, b_ref, lse_ref, m_acc, s_acc):
    i = pl.program_id(0)

    @pl.when(i == 0)
    def _():
        m_acc[...] = jnp.full_like(m_acc[...], _NEG)
        s_acc[...] = jnp.zeros_like(s_acc[...])

    logits = _logits_tile(emb_ref, w_ref, b_ref)
    col = i * _VT + lax.broadcasted_iota(jnp.int32, logits.shape, 1)
    logits = jnp.where(col < v, logits, _NEG)
    m_old = m_acc[...]
    m_new = jnp.maximum(m_old, jnp.max(logits, axis=1, keepdims=True))
    p_sum = jnp.sum(jnp.exp(logits - m_new), axis=1, keepdims=True)
    s_new = s_acc[...] * jnp.exp(m_old - m_new) + p_sum
    m_acc[...] = m_new
    s_acc[...] = s_new

    @pl.when(i == nt - 1)
    def _():
        lse_ref[...] = m_new + jnp.log(s_new)


def _out_body(b, v, nt, vtail, emb_ref, w_ref, b_ref, lse_ref, y_hbm,
              ybufs, sems):
    i = pl.program_id(0)
    rows = b // _K
    y = _logits_tile(emb_ref, w_ref, b_ref) - lse_ref[...]

    def stripes(nb, width):
        return [
            pltpu.make_async_copy(
                ybufs.at[nb, pl.ds(k * rows, rows), pl.ds(0, width)],
                y_hbm.at[pl.ds(k * rows, rows), pl.ds(i * _VT, width)],
                sems.at[nb, k],
            )
            for k in range(_K)
        ]

    def wait_prev(nb, width):
        for cp in stripes(nb, width):
            cp.wait()

    for nb in range(_NBUF):
        @pl.when(jnp.logical_and(i % _NBUF == nb, i >= _NBUF))
        def _(nb=nb):
            wait_prev(nb, _VT)

        @pl.when(jnp.logical_and(i % _NBUF == nb, i < nt - 1))
        def _(nb=nb):
            ybufs[nb] = y
            for cp in stripes(nb, _VT):
                cp.start()

        @pl.when(jnp.logical_and(i % _NBUF == nb, i == nt - 1))
        def _(nb=nb):
            ybufs[nb] = y
            for cp in stripes(nb, vtail):
                cp.start()
            for other in range(_NBUF):
                if other != nb:
                    wait_prev(other, _VT)
            for cp in stripes(nb, vtail):
                cp.wait()


def kernel(x, emb_table, W1, b1):
    b, c = x.shape
    v, e = emb_table.shape
    d = c * e
    n = b * c

    ep = max(e, 128)
    emb_pad = jnp.pad(emb_table, ((0, 0), (0, ep - e))) if ep != e else emb_table
    rows = _sc_gather(x.reshape(n).astype(jnp.int32), emb_pad)
    embeds = rows[:, :e].reshape(b, d)
    b2 = b1.reshape(1, v)
    nt = pl.cdiv(v, _VT)
    vrem = (v - (nt - 1) * _VT) % 128
    vtail = v - (nt - 1) * _VT - vrem

    lse = pl.pallas_call(
        functools.partial(_lse_body, v, nt),
        grid=(nt,),
        in_specs=[
            pl.BlockSpec((b, d), lambda i: (0, 0)),
            pl.BlockSpec((_VT, d), lambda i: (i, 0)),
            pl.BlockSpec((1, _VT), lambda i: (0, i)),
        ],
        out_specs=pl.BlockSpec((b, 1), lambda i: (0, 0)),
        out_shape=jax.ShapeDtypeStruct((b, 1), jnp.float32),
        scratch_shapes=[
            pltpu.VMEM((b, 1), jnp.float32),
            pltpu.VMEM((b, 1), jnp.float32),
        ],
        compiler_params=pltpu.CompilerParams(
            dimension_semantics=("arbitrary",)),
    )(embeds, W1, b2)

    y = pl.pallas_call(
        functools.partial(_out_body, b, v, nt, vtail),
        grid=(nt,),
        in_specs=[
            pl.BlockSpec((b, d), lambda i: (0, 0)),
            pl.BlockSpec((_VT, d), lambda i: (i, 0)),
            pl.BlockSpec((1, _VT), lambda i: (0, i)),
            pl.BlockSpec((b, 1), lambda i: (0, 0)),
        ],
        out_specs=pl.BlockSpec(memory_space=pltpu.MemorySpace.HBM),
        out_shape=jax.ShapeDtypeStruct((b, v), jnp.float32),
        scratch_shapes=[
            pltpu.VMEM((_NBUF, b, _VT), jnp.float32),
            pltpu.SemaphoreType.DMA((_NBUF, _K)),
        ],
        compiler_params=pltpu.CompilerParams(
            dimension_semantics=("arbitrary",)),
    )(embeds, W1, b2, lse)

    if vrem:
        ce = v - vrem
        tail = embeds @ W1[ce:, :].T + b1[ce:] - lse
        y = lax.dynamic_update_slice(y, tail, (0, ce))
    return y
